# baseline (speedup 1.0000x reference)
import functools

import jax
import jax.numpy as jnp
from jax import lax
from jax.experimental import pallas as pl
from jax.experimental.pallas import tpu as pltpu
from jax.experimental.pallas import tpu_sc as plsc

B, N, C, D, S, K = 8, 8192, 3, 128, 1024, 32
R = 128
SB = S // R
BS = B * S
BSK = BS * K
NW = 32
AW = BS // NW
A = 2
CK = A * K
NCHUNK = AW // A


NCH = 64
CL = N // NCH


def _tc_body(xyz_ref, fps_ref, nx_ref, idx_ref):
    b = pl.program_id(0)
    x = xyz_ref[0]
    fps = fps_ref[0, 0]
    oh = (lax.broadcasted_iota(jnp.int32, (N, R), 0) == fps[None, :]).astype(
        jnp.float32)
    a3 = lax.dot_general(x, oh, (((0,), (0,)), ((), ())),
                         preferred_element_type=jnp.float32)
    nx_ref[0] = a3
    dots2 = lax.dot_general(x, -2.0 * a3, (((1,), (0,)), ((), ())),
                            preferred_element_type=jnp.float32)
    a_sq = jnp.sum(a3 * a3, axis=0, keepdims=True)
    xsq = jnp.sum(x * x, axis=1, keepdims=True)
    d = (dots2 + a_sq) + xsq
    d4 = d.reshape(NCH, CL, R)
    M = jnp.min(d4, axis=1)
    i64 = lax.broadcasted_iota(jnp.int32, (NCH, R), 0)
    i128 = lax.broadcasted_iota(jnp.int32, (CL, R), 0)
    ci3 = lax.broadcasted_iota(jnp.int32, (NCH, 1, 1), 0)
    base = b * N
    inf = jnp.float32(jnp.inf)
    lastv = jnp.full((NCH, R), -jnp.inf, jnp.float32)
    lastl = jnp.full((NCH, R), -1, jnp.int32)
    for k in range(K):
        m = jnp.min(M, axis=0, keepdims=True)
        cmask = M == m
        cm = jnp.min(jnp.where(cmask, i64, NCH), axis=0,
                     keepdims=True)
        cmask = i64 == cm
        sel = jnp.min(jnp.where(ci3 == cm[:, None, :], d4, inf),
                      axis=0)
        lv = jnp.min(jnp.where(cmask, lastv, inf), axis=0,
                     keepdims=True)
        ll = jnp.max(jnp.where(cmask, lastl, -1), axis=0,
                     keepdims=True)
        sel = jnp.where((sel < lv) | ((sel == lv) & (i128 <= ll)), inf, sel)
        lm = jnp.min(jnp.where(sel == m, i128, CL), axis=0,
                     keepdims=True)
        idx_ref[0, pl.ds(k, 1), :] = cm * CL + lm + base
        newmin = jnp.min(jnp.where(i128 == lm, inf, sel), axis=0,
                         keepdims=True)
        M = jnp.where(cmask, newmin, M)
        lastv = jnp.where(cmask, m, lastv)
        lastl = jnp.where(cmask, lm, lastl)


def _tc_topk(xyz, fps3):
    grid = (B, SB)
    return pl.pallas_call(
        _tc_body,
        grid=grid,
        in_specs=[
            pl.BlockSpec((1, N, C), lambda b, s: (b, 0, 0)),
            pl.BlockSpec((1, 1, R), lambda b, s: (b * SB + s, 0, 0)),
        ],
        out_specs=[
            pl.BlockSpec((1, C, R), lambda b, s: (b * SB + s, 0, 0)),
            pl.BlockSpec((1, K, R), lambda b, s: (b * SB + s, 0, 0)),
        ],
        out_shape=[
            jax.ShapeDtypeStruct((B * SB, C, R), jnp.float32),
            jax.ShapeDtypeStruct((B * SB, K, R), jnp.int32),
        ],
    )(xyz, fps3)


def _sc_body(points_hbm, fps2_hbm, nidx2_hbm, out_hbm,
             fpsv, ctr, idxv0, idxv1, nbr0, nbr1, outv0, outv1,
             sem, gs0, gs1, ws0, ws1):
    wid = lax.axis_index("s") * 2 + lax.axis_index("c")
    idxvs, nbrs, outvs = (idxv0, idxv1), (nbr0, nbr1), (outv0, outv1)
    gss, wss = (gs0, gs1), (ws0, ws1)
    pltpu.sync_copy(fps2_hbm.at[pl.ds(wid * 2, 2)], fpsv)
    pltpu.async_copy(points_hbm.at[fpsv.at[0]], ctr.at[pl.ds(0, 128)],
                     sem).wait()
    pltpu.async_copy(points_hbm.at[fpsv.at[1]], ctr.at[pl.ds(128, 128)],
                     sem).wait()
    pltpu.sync_copy(nidx2_hbm.at[pl.ds(wid * NCHUNK, 1)], idxv0)
    pltpu.async_copy(points_hbm.at[idxv0.at[0]], nbr0, gs0)

    def outer(g, carry):
        for p in range(2):
            c = g * 2 + p
            q = 1 - p

            @pl.when(c + 1 < NCHUNK)
            def _():
                pltpu.sync_copy(nidx2_hbm.at[pl.ds(wid * NCHUNK + c + 1, 1)],
                                idxvs[q])
                pltpu.async_copy(points_hbm.at[idxvs[q].at[0]], nbrs[q],
                                 gss[q])

            pltpu.make_async_copy(points_hbm.at[idxvs[p].at[0]], nbrs[p],
                                  gss[p]).wait()

            @pl.when(c >= 2)
            def _():
                pltpu.make_async_copy(outvs[p], out_hbm.at[pl.ds(0, CK)],
                                      wss[p]).wait()

            @plsc.parallel_loop(0, CK, step=1, unroll=4)
            def _(r):
                crow = c * A + r // K
                for j in range(8):
                    cv = ctr[crow, pl.ds(j * 16, 16)]
                    nv = nbrs[p][r, pl.ds(j * 16, 16)]
                    outvs[p][r, pl.ds(j * 16, 16)] = nv - cv
                    outvs[p][r, pl.ds(D + j * 16, 16)] = cv
            pltpu.async_copy(outvs[p],
                             out_hbm.at[pl.ds((wid * NCHUNK + c) * CK, CK)],
                             wss[p])
        return carry

    lax.fori_loop(0, NCHUNK // 2, outer, 0)
    pltpu.make_async_copy(outv0, out_hbm.at[pl.ds(0, CK)], ws0).wait()
    pltpu.make_async_copy(outv1, out_hbm.at[pl.ds(0, CK)], ws1).wait()


def _sc_group(points_flat, fps2, nidx2):
    mesh = plsc.VectorSubcoreMesh(core_axis_name="c", subcore_axis_name="s")
    return pl.kernel(
        _sc_body,
        out_type=jax.ShapeDtypeStruct((BSK, 2 * D), jnp.float32),
        mesh=mesh,
        scratch_types=[
            pltpu.VMEM((2, 128), jnp.int32),
            pltpu.VMEM((AW, D), jnp.float32),
            pltpu.VMEM((1, CK), jnp.int32),
            pltpu.VMEM((1, CK), jnp.int32),
            pltpu.VMEM((CK, D), jnp.float32),
            pltpu.VMEM((CK, D), jnp.float32),
            pltpu.VMEM((CK, 2 * D), jnp.float32),
            pltpu.VMEM((CK, 2 * D), jnp.float32),
            pltpu.SemaphoreType.DMA,
            pltpu.SemaphoreType.DMA,
            pltpu.SemaphoreType.DMA,
            pltpu.SemaphoreType.DMA,
            pltpu.SemaphoreType.DMA,
        ],
    )(points_flat, fps2, nidx2)


@jax.jit
def kernel(xyz, points, fps_idx):
    fps3 = fps_idx.reshape(B * SB, 1, R)
    nx3, idxg = _tc_topk(xyz, fps3)
    new_xyz = jnp.swapaxes(nx3, 1, 2).reshape(B, S, C)
    idxg = jnp.swapaxes(idxg, 1, 2)
    points_flat = points.reshape(B * N, D)
    fps2 = (fps_idx + (jnp.arange(B, dtype=jnp.int32) * N)[:, None]).reshape(
        BS // 128, 128)
    nidx2 = idxg.reshape(BSK // CK, CK)
    out_flat = _sc_group(points_flat, fps2, nidx2)
    new_feats = out_flat.reshape(B, S, K, 2 * D)
    return new_xyz, new_feats


# speedup vs baseline: 1.0074x; 1.0074x over previous
import functools

import jax
import jax.numpy as jnp
from jax import lax
from jax.experimental import pallas as pl
from jax.experimental.pallas import tpu as pltpu
from jax.experimental.pallas import tpu_sc as plsc

B, N, C, D, S, K = 8, 8192, 3, 128, 1024, 32
R = 128
SB = S // R
BS = B * S
BSK = BS * K
NW = 32
AW = BS // NW
A = 2
CK = A * K
NCHUNK = AW // A


NCH = 64
CL = N // NCH


def _tc_body(xyz_ref, fps_ref, nx_ref, idx_ref):
    b = pl.program_id(0)
    x = xyz_ref[0]
    fps = fps_ref[0, 0]
    oh = (lax.broadcasted_iota(jnp.int32, (N, R), 0) == fps[None, :]).astype(
        jnp.float32)
    a3 = lax.dot_general(x, oh, (((0,), (0,)), ((), ())),
                         preferred_element_type=jnp.float32)
    nx_ref[0] = a3
    dots2 = lax.dot_general(x, -2.0 * a3, (((1,), (0,)), ((), ())),
                            preferred_element_type=jnp.float32)
    a_sq = jnp.sum(a3 * a3, axis=0, keepdims=True)
    xsq = jnp.sum(x * x, axis=1, keepdims=True)
    d = (dots2 + a_sq) + xsq
    d4 = d.reshape(NCH, CL, R)
    M = jnp.min(d4, axis=1)
    i64 = lax.broadcasted_iota(jnp.int32, (NCH, R), 0)
    i128 = lax.broadcasted_iota(jnp.int32, (CL, R), 0)
    ci3 = lax.broadcasted_iota(jnp.int32, (NCH, 1, 1), 0)
    base = b * N
    inf = jnp.float32(jnp.inf)
    lastv = jnp.full((NCH, R), -jnp.inf, jnp.float32)
    lastl = jnp.full((NCH, R), -1, jnp.int32)
    for k in range(K):
        m = jnp.min(M, axis=0, keepdims=True)
        cmask = M == m
        cm = jnp.min(jnp.where(cmask, i64, NCH), axis=0,
                     keepdims=True)
        cmask = i64 == cm
        sel = jnp.min(jnp.where(ci3 == cm[:, None, :], d4, inf),
                      axis=0)
        lv = jnp.min(jnp.where(cmask, lastv, inf), axis=0,
                     keepdims=True)
        ll = jnp.max(jnp.where(cmask, lastl, -1), axis=0,
                     keepdims=True)
        sel = jnp.where((sel < lv) | ((sel == lv) & (i128 <= ll)), inf, sel)
        lm = jnp.min(jnp.where(sel == m, i128, CL), axis=0,
                     keepdims=True)
        idx_ref[0, pl.ds(k, 1), :] = cm * CL + lm + base
        newmin = jnp.min(jnp.where(i128 == lm, inf, sel), axis=0,
                         keepdims=True)
        M = jnp.where(cmask, newmin, M)
        lastv = jnp.where(cmask, m, lastv)
        lastl = jnp.where(cmask, lm, lastl)


def _tc_topk(xyz, fps3):
    grid = (B, SB)
    return pl.pallas_call(
        _tc_body,
        grid=grid,
        in_specs=[
            pl.BlockSpec((1, N, C), lambda b, s: (b, 0, 0)),
            pl.BlockSpec((1, 1, R), lambda b, s: (b * SB + s, 0, 0)),
        ],
        out_specs=[
            pl.BlockSpec((1, C, R), lambda b, s: (b * SB + s, 0, 0)),
            pl.BlockSpec((1, K, R), lambda b, s: (b * SB + s, 0, 0)),
        ],
        out_shape=[
            jax.ShapeDtypeStruct((B * SB, C, R), jnp.float32),
            jax.ShapeDtypeStruct((B * SB, K, R), jnp.int32),
        ],
    )(xyz, fps3)


def _sc_body(points_hbm, fps2_hbm, nidx2_hbm, out_hbm,
             fpsv, ctr, idxv0, idxv1, nbr0, nbr1, outv0, outv1,
             sem, gs0, gs1, ws0, ws1):
    wid = lax.axis_index("s") * 2 + lax.axis_index("c")
    idxvs, nbrs, outvs = (idxv0, idxv1), (nbr0, nbr1), (outv0, outv1)
    gss, wss = (gs0, gs1), (ws0, ws1)
    pltpu.sync_copy(fps2_hbm.at[pl.ds(wid * 2, 2)], fpsv)
    pltpu.async_copy(points_hbm.at[fpsv.at[0]], ctr.at[pl.ds(0, 128)],
                     sem).wait()
    pltpu.async_copy(points_hbm.at[fpsv.at[1]], ctr.at[pl.ds(128, 128)],
                     sem).wait()
    pltpu.sync_copy(nidx2_hbm.at[pl.ds(wid * NCHUNK, 1)], idxv0)
    pltpu.async_copy(points_hbm.at[idxv0.at[0]], nbr0, gs0)

    def outer(g, carry):
        for p in range(2):
            c = g * 2 + p
            q = 1 - p

            @pl.when(c + 1 < NCHUNK)
            def _():
                pltpu.sync_copy(nidx2_hbm.at[pl.ds(wid * NCHUNK + c + 1, 1)],
                                idxvs[q])
                pltpu.async_copy(points_hbm.at[idxvs[q].at[0]], nbrs[q],
                                 gss[q])

            pltpu.make_async_copy(points_hbm.at[idxvs[p].at[0]], nbrs[p],
                                  gss[p]).wait()

            @pl.when(c >= 2)
            def _():
                pltpu.make_async_copy(outvs[p], out_hbm.at[pl.ds(0, CK)],
                                      wss[p]).wait()

            @plsc.parallel_loop(0, CK, step=1, unroll=2)
            def _(r):
                crow = c * A + r // K
                for j in range(8):
                    cv = ctr[crow, pl.ds(j * 16, 16)]
                    nv = nbrs[p][r, pl.ds(j * 16, 16)]
                    outvs[p][r, pl.ds(j * 16, 16)] = nv - cv
                    outvs[p][r, pl.ds(D + j * 16, 16)] = cv
            pltpu.async_copy(outvs[p],
                             out_hbm.at[pl.ds((wid * NCHUNK + c) * CK, CK)],
                             wss[p])
        return carry

    lax.fori_loop(0, NCHUNK // 2, outer, 0)
    pltpu.make_async_copy(outv0, out_hbm.at[pl.ds(0, CK)], ws0).wait()
    pltpu.make_async_copy(outv1, out_hbm.at[pl.ds(0, CK)], ws1).wait()


def _sc_group(points_flat, fps2, nidx2):
    mesh = plsc.VectorSubcoreMesh(core_axis_name="c", subcore_axis_name="s")
    return pl.kernel(
        _sc_body,
        out_type=jax.ShapeDtypeStruct((BSK, 2 * D), jnp.float32),
        mesh=mesh,
        scratch_types=[
            pltpu.VMEM((2, 128), jnp.int32),
            pltpu.VMEM((AW, D), jnp.float32),
            pltpu.VMEM((1, CK), jnp.int32),
            pltpu.VMEM((1, CK), jnp.int32),
            pltpu.VMEM((CK, D), jnp.float32),
            pltpu.VMEM((CK, D), jnp.float32),
            pltpu.VMEM((CK, 2 * D), jnp.float32),
            pltpu.VMEM((CK, 2 * D), jnp.float32),
            pltpu.SemaphoreType.DMA,
            pltpu.SemaphoreType.DMA,
            pltpu.SemaphoreType.DMA,
            pltpu.SemaphoreType.DMA,
            pltpu.SemaphoreType.DMA,
        ],
    )(points_flat, fps2, nidx2)


@jax.jit
def kernel(xyz, points, fps_idx):
    fps3 = fps_idx.reshape(B * SB, 1, R)
    nx3, idxg = _tc_topk(xyz, fps3)
    new_xyz = jnp.swapaxes(nx3, 1, 2).reshape(B, S, C)
    idxg = jnp.swapaxes(idxg, 1, 2)
    points_flat = points.reshape(B * N, D)
    fps2 = (fps_idx + (jnp.arange(B, dtype=jnp.int32) * N)[:, None]).reshape(
        BS // 128, 128)
    nidx2 = idxg.reshape(BSK // CK, CK)
    out_flat = _sc_group(points_flat, fps2, nidx2)
    new_feats = out_flat.reshape(B, S, K, 2 * D)
    return new_xyz, new_feats
